# Initial kernel scaffold; baseline (speedup 1.0000x reference)
import jax
import jax.numpy as jnp
from jax.experimental import pallas as pl


def kernel(z, embedding_w):
    raise NotImplementedError("write your pallas kernel here")


# speedup vs baseline: 1.0769x; 1.0769x over previous
import functools

import jax
import jax.numpy as jnp
from jax import lax
from jax.experimental import pallas as pl
from jax.experimental.pallas import tpu as pltpu
from jax.experimental.pallas import tpu_sc as plsc

N_E = 8192
E_DIM = 256
BETA = 0.25

TB = 256
NB = 1024
N_TOK = 8192
N_STEPS = N_TOK // TB
N_CHUNKS = N_E // NB
SEG = 4096


def _dist_argmin_body(z_ref, e_ref, idx_ref, oh_ref, perp_ref,
                      esq_ref, counts_ref):
    i = pl.program_id(0)

    zb = z_ref[...]
    zsq = jnp.sum(zb * zb, axis=1)

    @pl.when(i == 0)
    def _():
        for c in range(N_CHUNKS):
            ec = e_ref[pl.ds(c * NB, NB), :]
            esq_ref[0, pl.ds(c * NB, NB)] = jnp.sum(ec * ec, axis=1)

    n_seg = N_E // SEG
    cpseg = SEG // NB
    rmin = [jnp.full((TB,), jnp.inf, dtype=jnp.float32) for _ in range(n_seg)]
    rarg = [jnp.zeros((TB,), dtype=jnp.int32) for _ in range(n_seg)]
    for c in range(N_CHUNKS):
        ec = e_ref[pl.ds(c * NB, NB), :]
        m = lax.dot_general(zb, ec, (((1,), (1,)), ((), ())),
                            preferred_element_type=jnp.float32)
        esq_c = esq_ref[0, pl.ds(c * NB, NB)]
        d = (zsq[:, None] + esq_c[None, :]) - 2.0 * m
        bm = jnp.min(d, axis=1)
        jj = lax.broadcasted_iota(jnp.int32, (TB, NB), 1) + c * NB
        cand = jnp.where(d == bm[:, None], jj, jnp.int32(2 ** 30))
        barg = jnp.min(cand, axis=1)
        k = c // cpseg
        better = bm < rmin[k]
        rarg[k] = jnp.where(better, barg, rarg[k])
        rmin[k] = jnp.where(better, bm, rmin[k])

    accv = rmin[0].astype(jnp.bfloat16).astype(jnp.float32)
    acci = rarg[0]
    for k in range(1, n_seg):
        wk = rmin[k] < accv
        acci = jnp.where(wk, rarg[k], acci)
        accv = jnp.where(wk, rmin[k], accv).astype(jnp.bfloat16).astype(jnp.float32)
    rarg_final = acci

    idx_ref[0, 0, :] = rarg_final

    lane = lax.broadcasted_iota(jnp.int32, (TB, N_E), 1)
    oh = jnp.where(lane == rarg_final[:, None], jnp.float32(1.0), jnp.float32(0.0))
    oh_ref[...] = oh

    csum = jnp.sum(oh, axis=0)[None, :]

    @pl.when(i == 0)
    def _():
        counts_ref[...] = csum

    @pl.when(i > 0)
    def _():
        counts_ref[...] = counts_ref[...] + csum

    @pl.when(i == N_STEPS - 1)
    def _():
        e_mean = counts_ref[...] * jnp.float32(1.0 / N_TOK)
        ent = jnp.sum(e_mean * jnp.log(e_mean + 1e-10), keepdims=True)
        perp_ref[...] = jnp.exp(-ent)


def _dist_argmin(zf, ew):
    return pl.pallas_call(
        _dist_argmin_body,
        grid=(N_STEPS,),
        in_specs=[
            pl.BlockSpec((TB, E_DIM), lambda i: (i, 0)),
            pl.BlockSpec((N_E, E_DIM), lambda i: (0, 0)),
        ],
        out_specs=[
            pl.BlockSpec((1, 1, TB), lambda i: (i, 0, 0)),
            pl.BlockSpec((TB, N_E), lambda i: (i, 0)),
            pl.BlockSpec((1, 1), lambda i: (0, 0)),
        ],
        out_shape=[
            jax.ShapeDtypeStruct((N_STEPS, 1, TB), jnp.int32),
            jax.ShapeDtypeStruct((N_TOK, N_E), jnp.float32),
            jax.ShapeDtypeStruct((1, 1), jnp.float32),
        ],
        scratch_shapes=[
            pltpu.VMEM((1, N_E), jnp.float32),
            pltpu.VMEM((1, N_E), jnp.float32),
        ],
    )(zf, ew)


@functools.lru_cache(maxsize=1)
def _make_sc_gather():
    info = plsc.get_sparse_core_info()
    nc, ns = info.num_cores, info.num_subcores
    nw = nc * ns
    b_per_w = N_TOK // nw
    mesh = plsc.VectorSubcoreMesh(core_axis_name="c", subcore_axis_name="s")

    @functools.partial(
        pl.kernel, mesh=mesh,
        out_type=jax.ShapeDtypeStruct((N_TOK, E_DIM), jnp.float32),
        scratch_types=[
            pltpu.VMEM((b_per_w,), jnp.int32),
            pltpu.VMEM((b_per_w, E_DIM), jnp.float32),
            pltpu.SemaphoreType.DMA,
        ],
    )
    def gather_rows(table_hbm, idx_hbm, out_hbm, idx_v, rows_v, sem):
        wid = lax.axis_index("s") * nc + lax.axis_index("c")
        base = wid * b_per_w
        pltpu.sync_copy(idx_hbm.at[pl.ds(base, b_per_w)], idx_v)
        pltpu.async_copy(table_hbm.at[idx_v], rows_v, sem).wait()
        pltpu.sync_copy(rows_v, out_hbm.at[pl.ds(base, b_per_w)])

    return gather_rows


def _loss_zq_body(z_ref, zq1_ref, zq_ref, loss_ref, acc_ref):
    i = pl.program_id(0)
    zb = z_ref[...]
    qb = zq1_ref[...]
    diff = qb - zb
    zq_ref[...] = zb + diff
    s = jnp.sum(diff * diff)

    @pl.when(i == 0)
    def _():
        acc_ref[0, 0] = s

    @pl.when(i > 0)
    def _():
        acc_ref[0, 0] = acc_ref[0, 0] + s

    @pl.when(i == N_STEPS - 1)
    def _():
        mse = acc_ref[0, 0] * jnp.float32(1.0 / (N_TOK * E_DIM))
        l = mse + jnp.float32(BETA) * mse
        loss_ref[...] = jnp.full((1, 1), l, dtype=jnp.float32)


def _loss_zq(zf, zq1f):
    return pl.pallas_call(
        _loss_zq_body,
        grid=(N_STEPS,),
        in_specs=[
            pl.BlockSpec((TB, E_DIM), lambda i: (i, 0)),
            pl.BlockSpec((TB, E_DIM), lambda i: (i, 0)),
        ],
        out_specs=[
            pl.BlockSpec((TB, E_DIM), lambda i: (i, 0)),
            pl.BlockSpec((1, 1), lambda i: (0, 0)),
        ],
        out_shape=[
            jax.ShapeDtypeStruct((N_TOK, E_DIM), jnp.float32),
            jax.ShapeDtypeStruct((1, 1), jnp.float32),
        ],
        scratch_shapes=[pltpu.SMEM((1, 1), jnp.float32)],
    )(zf, zq1f)


def kernel(z, embedding_w):
    b, ch, h, w = z.shape
    zp_shape = (b, h, w, ch)
    zf = jnp.transpose(z, (0, 2, 3, 1)).reshape(N_TOK, E_DIM)

    idx3, min_encodings, perp = _dist_argmin(zf, embedding_w)
    idx = idx3.reshape(N_TOK)

    zq1f = _make_sc_gather()(embedding_w, idx)
    zq1f = zq1f.astype(jnp.bfloat16).astype(jnp.float32)

    zqf, loss2 = _loss_zq(zf, zq1f)

    loss = loss2[0, 0]
    perplexity = perp[0, 0]
    min_encoding_indices = idx.reshape(N_TOK, 1)
    z_q = jnp.transpose(zqf.reshape(zp_shape), (0, 3, 1, 2))
    z_q1_out = jnp.transpose(zq1f.reshape(zp_shape), (0, 3, 1, 2))
    return (loss, z_q, perplexity, min_encodings, min_encoding_indices,
            embedding_w, z_q1_out)
